# baseline (speedup 1.0000x reference)
import functools

import jax
import jax.numpy as jnp
from jax import lax
from jax.experimental import pallas as pl
from jax.experimental.pallas import tpu as pltpu
from jax.experimental.pallas import tpu_sc as plsc

B = 16384
D = 64
L = 16
NF = 16
NG = D // NF
NB = 7812
NMAIN = NB * 128
FLATG = NF * NMAIN
TILE_STRIDE = NB * 1024

_info = plsc.get_sparse_core_info()
_NC = _info.num_cores
_NW = _info.num_cores * _info.num_subcores
BPW = B // _NW
CH = 256
NCH = BPW // CH
NBLK = CH // L
NSLOT = 2


def _make_dot_body(k0):
    def _dot_body(da_hbm, db_hbm, flat_hbm, tail_hbm, w_hbm, out_hbm,
                  idx_a, idx_b, ga, gb, ra, rb, tail_v, w_v, out_v, sems_a,
                  sems_b):
        wid = lax.axis_index("s") * _NC + lax.axis_index("c")
        base = wid * BPW

        pltpu.sync_copy(da_hbm.at[pl.ds(base, BPW)], idx_a)
        pltpu.sync_copy(db_hbm.at[pl.ds(base, BPW)], idx_b)
        pltpu.sync_copy(w_hbm, w_v)
        pltpu.sync_copy(tail_hbm, tail_v)

        lane = jnp.arange(L, dtype=jnp.int32)

        def _splat(vec, j):
            idx = jnp.full((L, 1), j, jnp.int32)
            dnums = lax.GatherDimensionNumbers(
                offset_dims=(), collapsed_slice_dims=(0,),
                start_index_map=(0,))
            return lax.gather(vec, idx, dnums, (1,),
                              mode=lax.GatherScatterMode.PROMISE_IN_BOUNDS)

        wchunk = w_v[pl.ds(k0, L)]
        wsp = [_splat(wchunk, j) for j in range(NF)]

        kconst = [(k >> 3) * TILE_STRIDE + (k & 7) * 128 for k in range(NF)]

        def fill_chunk(coff, slot):
            def idx_body(blk, carry):
                off = pl.multiple_of(blk * L, L)
                ia = jnp.minimum(idx_a[pl.ds(coff + off, L)], NMAIN - 1)
                ib = jnp.minimum(idx_b[pl.ds(coff + off, L)], NMAIN - 1)
                basea = lax.shift_right_logical(ia, 7) * 1024 + (ia & 127)
                baseb = lax.shift_right_logical(ib, 7) * 1024 + (ib & 127)
                for k in range(NF):
                    ga[slot][pl.ds(k * CH + off, L)] = basea + kconst[k]
                    gb[slot][pl.ds(k * CH + off, L)] = baseb + kconst[k]
                return carry
            lax.fori_loop(0, NBLK, idx_body, 0)

        def fire_chunk(slot):
            cpa = pltpu.async_copy(
                flat_hbm.at[ga[slot]], ra[slot], sems_a[slot])
            cpb = pltpu.async_copy(
                flat_hbm.at[gb[slot]], rb[slot], sems_b[slot])
            return cpa, cpb

        def process_chunk(coff, rabuf, rbbuf):
            def fix_body(blk, carry):
                off = pl.multiple_of(blk * L, L)

                def fix_one(idx_ref, rows_ref):
                    ii = idx_ref[pl.ds(coff + off, L)]
                    mask = ii >= NMAIN
                    anyt = lax.reduce_max(mask.astype(jnp.int32), axes=(0,))

                    @pl.when(anyt > 0)
                    def _():
                        rowt = jnp.clip(ii - NMAIN, 0, D - 1)
                        for k in range(NF):
                            tv = plsc.load_gather(
                                tail_v,
                                [rowt, jnp.full((L,), k0 + k, jnp.int32)])
                            plsc.store_scatter(
                                rows_ref, [k * CH + off + lane], tv,
                                mask=mask)

                fix_one(idx_a, rabuf)
                fix_one(idx_b, rbbuf)
                return carry

            lax.fori_loop(0, NBLK, fix_body, 0)

            def blk_body(blk, carry):
                off = pl.multiple_of(blk * L, L)
                acc = jnp.zeros((L,), jnp.float32)
                for k in range(NF):
                    av = rabuf[pl.ds(k * CH + off, L)]
                    bv = rbbuf[pl.ds(k * CH + off, L)]
                    acc = acc + av * bv * wsp[k]
                out_v[pl.ds(coff + off, L)] = acc
                return carry

            lax.fori_loop(0, NBLK, blk_body, 0)

        inflight = {}
        for s in range(min(NSLOT, NCH)):
            fill_chunk(s * CH, s)
            inflight[s] = fire_chunk(s)
        for c in range(NCH):
            slot = c % NSLOT
            cpa, cpb = inflight[slot]
            cpa.wait()
            cpb.wait()
            process_chunk(c * CH, ra[slot], rb[slot])
            nxt = c + NSLOT
            if nxt < NCH:
                fill_chunk(nxt * CH, slot)
                inflight[slot] = fire_chunk(slot)

        pltpu.sync_copy(out_v, out_hbm.at[pl.ds(base, BPW)])

    return _dot_body


def _bias_body(p0_hbm, p1_hbm, p2_hbm, p3_hbm, da_hbm, db_hbm, bias_hbm,
               tailb_hbm, w_hbm, b0_hbm, out_hbm, p0_v, p1_v, p2_v, p3_v,
               idx_a, idx_b, ic_a, ic_b, bia, bib, tailb_v, w_v, b0_v,
               out_v, sem):
    wid = lax.axis_index("s") * _NC + lax.axis_index("c")
    base = wid * BPW

    pltpu.sync_copy(p0_hbm.at[pl.ds(base, BPW)], p0_v)
    pltpu.sync_copy(p1_hbm.at[pl.ds(base, BPW)], p1_v)
    pltpu.sync_copy(p2_hbm.at[pl.ds(base, BPW)], p2_v)
    pltpu.sync_copy(p3_hbm.at[pl.ds(base, BPW)], p3_v)
    pltpu.sync_copy(da_hbm.at[pl.ds(base, BPW)], idx_a)
    pltpu.sync_copy(db_hbm.at[pl.ds(base, BPW)], idx_b)
    pltpu.sync_copy(w_hbm, w_v)
    pltpu.sync_copy(b0_hbm, b0_v)
    pltpu.sync_copy(tailb_hbm, tailb_v)

    def clamp_body(blk, carry):
        off = pl.multiple_of(blk * L, L)
        ic_a[pl.ds(off, L)] = jnp.minimum(idx_a[pl.ds(off, L)], NMAIN - 1)
        ic_b[pl.ds(off, L)] = jnp.minimum(idx_b[pl.ds(off, L)], NMAIN - 1)
        return carry

    lax.fori_loop(0, BPW // L, clamp_body, 0)

    pltpu.async_copy(bias_hbm.at[ic_a], bia, sem).wait()
    pltpu.async_copy(bias_hbm.at[ic_b], bib, sem).wait()

    def _splat(vec, j):
        idx = jnp.full((L, 1), j, jnp.int32)
        dnums = lax.GatherDimensionNumbers(
            offset_dims=(), collapsed_slice_dims=(0,), start_index_map=(0,))
        return lax.gather(vec, idx, dnums, (1,),
                          mode=lax.GatherScatterMode.PROMISE_IN_BOUNDS)

    w_last = _splat(w_v[pl.ds(D, L)], 0)
    b0_vec = _splat(b0_v[pl.ds(0, L)], 0)

    def blk_body(blk, carry):
        off = pl.multiple_of(blk * L, L)
        ba = bia[pl.ds(off, L)]
        bb = bib[pl.ds(off, L)]

        def fix_one(idx_ref, bv):
            ii = idx_ref[pl.ds(off, L)]
            mask = ii >= NMAIN
            rowt = jnp.clip(ii - NMAIN, 0, D - 1)
            tb = plsc.load_gather(tailb_v, [rowt])
            return jnp.where(mask, tb, bv)

        ba = fix_one(idx_a, ba)
        bb = fix_one(idx_b, bb)
        acc = (p0_v[pl.ds(off, L)] + p1_v[pl.ds(off, L)]
               + p2_v[pl.ds(off, L)] + p3_v[pl.ds(off, L)]
               + (ba + bb) * w_last + b0_vec)
        out_v[pl.ds(off, L)] = 1.0 / (1.0 + jnp.exp(-acc))
        return carry

    lax.fori_loop(0, BPW // L, blk_body, 0)

    pltpu.sync_copy(out_v, out_hbm.at[pl.ds(base, BPW)])


def _mk_mesh():
    return plsc.VectorSubcoreMesh(core_axis_name="c", subcore_axis_name="s")


@jax.jit
def _afmp(da, db, flats, tail, bias_flat, tail_bias, dense_w, dense_b):
    parts = []
    for g in range(NG):
        k2 = functools.partial(
            pl.kernel,
            mesh=_mk_mesh(),
            compiler_params=pltpu.CompilerParams(needs_layout_passes=False),
            out_type=jax.ShapeDtypeStruct((B,), jnp.float32),
            scratch_types=[
                pltpu.VMEM((BPW,), jnp.int32),
                pltpu.VMEM((BPW,), jnp.int32),
                [pltpu.VMEM((NF * CH,), jnp.int32) for _ in range(NSLOT)],
                [pltpu.VMEM((NF * CH,), jnp.int32) for _ in range(NSLOT)],
                [pltpu.VMEM((NF * CH,), jnp.float32) for _ in range(NSLOT)],
                [pltpu.VMEM((NF * CH,), jnp.float32) for _ in range(NSLOT)],
                pltpu.VMEM((D, D), jnp.float32),
                pltpu.VMEM((D + L,), jnp.float32),
                pltpu.VMEM((BPW,), jnp.float32),
                [pltpu.SemaphoreType.DMA for _ in range(NSLOT)],
                [pltpu.SemaphoreType.DMA for _ in range(NSLOT)],
            ],
        )(_make_dot_body(g * NF))
        parts.append(k2(da, db, flats[g], tail, dense_w))

    k3 = functools.partial(
        pl.kernel,
        mesh=_mk_mesh(),
        compiler_params=pltpu.CompilerParams(needs_layout_passes=False),
        out_type=jax.ShapeDtypeStruct((B,), jnp.float32),
        scratch_types=[
            pltpu.VMEM((BPW,), jnp.float32),
            pltpu.VMEM((BPW,), jnp.float32),
            pltpu.VMEM((BPW,), jnp.float32),
            pltpu.VMEM((BPW,), jnp.float32),
            pltpu.VMEM((BPW,), jnp.int32),
            pltpu.VMEM((BPW,), jnp.int32),
            pltpu.VMEM((BPW,), jnp.int32),
            pltpu.VMEM((BPW,), jnp.int32),
            pltpu.VMEM((BPW,), jnp.float32),
            pltpu.VMEM((BPW,), jnp.float32),
            pltpu.VMEM((D,), jnp.float32),
            pltpu.VMEM((D + L,), jnp.float32),
            pltpu.VMEM((L,), jnp.float32),
            pltpu.VMEM((BPW,), jnp.float32),
            pltpu.SemaphoreType.DMA,
        ],
    )(_bias_body)
    return k3(parts[0], parts[1], parts[2], parts[3], da, db, bias_flat,
              tail_bias, dense_w, dense_b)


def kernel(drug_a, drug_b, emb_table, bias_table, dense_w, dense_b):
    da = drug_a.astype(jnp.int32)
    db = drug_b.astype(jnp.int32)
    emb_t = emb_table.T
    flats = []
    for g in range(NG):
        flats.append(emb_t[g * NF:(g + 1) * NF, :NMAIN]
                     .reshape(NF // 8, 8, NB, 128)
                     .transpose(0, 2, 1, 3).reshape(FLATG))
    lastflat, bias_table = lax.optimization_barrier((flats[-1], bias_table))
    flats[-1] = lastflat
    bias_flat = bias_table.T[:, :NMAIN].reshape(NMAIN)
    tail_bias = bias_table.T[:, NMAIN:NMAIN + D].reshape(D)
    tail = emb_table[NMAIN:NMAIN + D]
    w_pad = jnp.pad(dense_w.reshape(-1), (0, L - 1))
    b0_pad = jnp.pad(dense_b, (0, L - 1))
    out = _afmp(da, db, flats, tail, bias_flat, tail_bias, w_pad, b0_pad)
    return out.reshape(B, 1)


# speedup vs baseline: 1.1264x; 1.1264x over previous
import functools

import jax
import jax.numpy as jnp
from jax import lax
from jax.experimental import pallas as pl
from jax.experimental.pallas import tpu as pltpu
from jax.experimental.pallas import tpu_sc as plsc

B = 16384
D = 64
L = 16
NF = 16
NG = D // NF
NB = 7812
NMAIN = NB * 128
FLATG = NF * NMAIN
TILE_STRIDE = NB * 1024

_info = plsc.get_sparse_core_info()
_NC = _info.num_cores
_NW = _info.num_cores * _info.num_subcores
BPW = B // _NW
CH = 256
NCH = BPW // CH
NBLK = CH // L
NSLOT = 2


def _make_dot_body(k0):
    def _dot_body(da_hbm, db_hbm, flat_hbm, tail_hbm, w_hbm, out_hbm,
                  idx_a, idx_b, ga, gb, ra, rb, tail_v, w_v, out_v, sems_a,
                  sems_b):
        wid = lax.axis_index("s") * _NC + lax.axis_index("c")
        base = wid * BPW

        pltpu.sync_copy(da_hbm.at[pl.ds(base, BPW)], idx_a)
        pltpu.sync_copy(db_hbm.at[pl.ds(base, BPW)], idx_b)
        pltpu.sync_copy(w_hbm, w_v)
        pltpu.sync_copy(tail_hbm, tail_v)

        lane = jnp.arange(L, dtype=jnp.int32)

        def _splat(vec, j):
            idx = jnp.full((L, 1), j, jnp.int32)
            dnums = lax.GatherDimensionNumbers(
                offset_dims=(), collapsed_slice_dims=(0,),
                start_index_map=(0,))
            return lax.gather(vec, idx, dnums, (1,),
                              mode=lax.GatherScatterMode.PROMISE_IN_BOUNDS)

        wchunk = w_v[pl.ds(k0, L)]
        wsp = [_splat(wchunk, j) for j in range(NF)]

        kconst = [(k >> 3) * TILE_STRIDE + (k & 7) * 128 for k in range(NF)]

        def fill_chunk(coff, slot):
            def idx_body(blk, carry):
                off = pl.multiple_of(blk * L, L)
                ia = jnp.minimum(idx_a[pl.ds(coff + off, L)], NMAIN - 1)
                ib = jnp.minimum(idx_b[pl.ds(coff + off, L)], NMAIN - 1)
                basea = lax.shift_right_logical(ia, 7) * 1024 + (ia & 127)
                baseb = lax.shift_right_logical(ib, 7) * 1024 + (ib & 127)
                for k in range(NF):
                    ga[slot][pl.ds(k * CH + off, L)] = basea + kconst[k]
                    gb[slot][pl.ds(k * CH + off, L)] = baseb + kconst[k]
                return carry
            lax.fori_loop(0, NBLK, idx_body, 0)

        def fire_chunk(slot):
            cpa = pltpu.async_copy(
                flat_hbm.at[ga[slot]], ra[slot], sems_a[slot])
            cpb = pltpu.async_copy(
                flat_hbm.at[gb[slot]], rb[slot], sems_b[slot])
            return cpa, cpb

        def process_chunk(coff, rabuf, rbbuf):
            def fix_body(blk, carry):
                off = pl.multiple_of(blk * L, L)

                def fix_one(idx_ref, rows_ref):
                    ii = idx_ref[pl.ds(coff + off, L)]
                    mask = ii >= NMAIN
                    anyt = lax.reduce_max(mask.astype(jnp.int32), axes=(0,))

                    @pl.when(anyt > 0)
                    def _():
                        rowt = jnp.clip(ii - NMAIN, 0, D - 1)
                        for k in range(NF):
                            tv = plsc.load_gather(
                                tail_v,
                                [rowt, jnp.full((L,), k0 + k, jnp.int32)])
                            plsc.store_scatter(
                                rows_ref, [k * CH + off + lane], tv,
                                mask=mask)

                fix_one(idx_a, rabuf)
                fix_one(idx_b, rbbuf)
                return carry

            lax.fori_loop(0, NBLK, fix_body, 0)

            def blk_body(blk, carry):
                off = pl.multiple_of(blk * L, L)
                acc = jnp.zeros((L,), jnp.float32)
                for k in range(NF):
                    av = rabuf[pl.ds(k * CH + off, L)]
                    bv = rbbuf[pl.ds(k * CH + off, L)]
                    acc = acc + av * bv * wsp[k]
                out_v[pl.ds(coff + off, L)] = acc
                return carry

            lax.fori_loop(0, NBLK, blk_body, 0)

        inflight = {}
        for s in range(min(NSLOT, NCH)):
            fill_chunk(s * CH, s)
            inflight[s] = fire_chunk(s)
        for c in range(NCH):
            slot = c % NSLOT
            cpa, cpb = inflight[slot]
            cpa.wait()
            cpb.wait()
            process_chunk(c * CH, ra[slot], rb[slot])
            nxt = c + NSLOT
            if nxt < NCH:
                fill_chunk(nxt * CH, slot)
                inflight[slot] = fire_chunk(slot)

        pltpu.sync_copy(out_v, out_hbm.at[pl.ds(base, BPW)])

    return _dot_body


def _bias_body(p0_hbm, p1_hbm, p2_hbm, p3_hbm, da_hbm, db_hbm, bias_hbm,
               tailb_hbm, w_hbm, b0_hbm, out_hbm, p0_v, p1_v, p2_v, p3_v,
               idx_a, idx_b, ic_a, ic_b, bia, bib, tailb_v, w_v, b0_v,
               out_v, sem):
    wid = lax.axis_index("s") * _NC + lax.axis_index("c")
    base = wid * BPW

    pltpu.sync_copy(p0_hbm.at[pl.ds(base, BPW)], p0_v)
    pltpu.sync_copy(p1_hbm.at[pl.ds(base, BPW)], p1_v)
    pltpu.sync_copy(p2_hbm.at[pl.ds(base, BPW)], p2_v)
    pltpu.sync_copy(p3_hbm.at[pl.ds(base, BPW)], p3_v)
    pltpu.sync_copy(da_hbm.at[pl.ds(base, BPW)], idx_a)
    pltpu.sync_copy(db_hbm.at[pl.ds(base, BPW)], idx_b)
    pltpu.sync_copy(w_hbm, w_v)
    pltpu.sync_copy(b0_hbm, b0_v)
    pltpu.sync_copy(tailb_hbm, tailb_v)

    def clamp_body(blk, carry):
        off = pl.multiple_of(blk * L, L)
        ic_a[pl.ds(off, L)] = jnp.minimum(idx_a[pl.ds(off, L)], NMAIN - 1)
        ic_b[pl.ds(off, L)] = jnp.minimum(idx_b[pl.ds(off, L)], NMAIN - 1)
        return carry

    lax.fori_loop(0, BPW // L, clamp_body, 0)

    pltpu.async_copy(bias_hbm.at[ic_a], bia, sem).wait()
    pltpu.async_copy(bias_hbm.at[ic_b], bib, sem).wait()

    def _splat(vec, j):
        idx = jnp.full((L, 1), j, jnp.int32)
        dnums = lax.GatherDimensionNumbers(
            offset_dims=(), collapsed_slice_dims=(0,), start_index_map=(0,))
        return lax.gather(vec, idx, dnums, (1,),
                          mode=lax.GatherScatterMode.PROMISE_IN_BOUNDS)

    w_last = _splat(w_v[pl.ds(D, L)], 0)
    b0_vec = _splat(b0_v[pl.ds(0, L)], 0)

    def blk_body(blk, carry):
        off = pl.multiple_of(blk * L, L)
        ba = bia[pl.ds(off, L)]
        bb = bib[pl.ds(off, L)]

        def fix_one(idx_ref, bv):
            ii = idx_ref[pl.ds(off, L)]
            mask = ii >= NMAIN
            rowt = jnp.clip(ii - NMAIN, 0, D - 1)
            tb = plsc.load_gather(tailb_v, [rowt])
            return jnp.where(mask, tb, bv)

        ba = fix_one(idx_a, ba)
        bb = fix_one(idx_b, bb)
        acc = (p0_v[pl.ds(off, L)] + p1_v[pl.ds(off, L)]
               + p2_v[pl.ds(off, L)] + p3_v[pl.ds(off, L)]
               + (ba + bb) * w_last + b0_vec)
        out_v[pl.ds(off, L)] = 1.0 / (1.0 + jnp.exp(-acc))
        return carry

    lax.fori_loop(0, BPW // L, blk_body, 0)

    pltpu.sync_copy(out_v, out_hbm.at[pl.ds(base, BPW)])


def _mk_mesh():
    return plsc.VectorSubcoreMesh(core_axis_name="c", subcore_axis_name="s")


@jax.jit
def _afmp(da, db, flats, tail, bias_flat, tail_bias, dense_w, dense_b):
    parts = []
    for g in range(NG):
        k2 = functools.partial(
            pl.kernel,
            mesh=_mk_mesh(),
            compiler_params=pltpu.CompilerParams(needs_layout_passes=False),
            out_type=jax.ShapeDtypeStruct((B,), jnp.float32),
            scratch_types=[
                pltpu.VMEM((BPW,), jnp.int32),
                pltpu.VMEM((BPW,), jnp.int32),
                [pltpu.VMEM((NF * CH,), jnp.int32) for _ in range(NSLOT)],
                [pltpu.VMEM((NF * CH,), jnp.int32) for _ in range(NSLOT)],
                [pltpu.VMEM((NF * CH,), jnp.float32) for _ in range(NSLOT)],
                [pltpu.VMEM((NF * CH,), jnp.float32) for _ in range(NSLOT)],
                pltpu.VMEM((D, D), jnp.float32),
                pltpu.VMEM((D + L,), jnp.float32),
                pltpu.VMEM((BPW,), jnp.float32),
                [pltpu.SemaphoreType.DMA for _ in range(NSLOT)],
                [pltpu.SemaphoreType.DMA for _ in range(NSLOT)],
            ],
        )(_make_dot_body(g * NF))
        parts.append(k2(da, db, flats[g], tail, dense_w))

    k3 = functools.partial(
        pl.kernel,
        mesh=_mk_mesh(),
        compiler_params=pltpu.CompilerParams(needs_layout_passes=False),
        out_type=jax.ShapeDtypeStruct((B,), jnp.float32),
        scratch_types=[
            pltpu.VMEM((BPW,), jnp.float32),
            pltpu.VMEM((BPW,), jnp.float32),
            pltpu.VMEM((BPW,), jnp.float32),
            pltpu.VMEM((BPW,), jnp.float32),
            pltpu.VMEM((BPW,), jnp.int32),
            pltpu.VMEM((BPW,), jnp.int32),
            pltpu.VMEM((BPW,), jnp.int32),
            pltpu.VMEM((BPW,), jnp.int32),
            pltpu.VMEM((BPW,), jnp.float32),
            pltpu.VMEM((BPW,), jnp.float32),
            pltpu.VMEM((D,), jnp.float32),
            pltpu.VMEM((D + L,), jnp.float32),
            pltpu.VMEM((L,), jnp.float32),
            pltpu.VMEM((BPW,), jnp.float32),
            pltpu.SemaphoreType.DMA,
        ],
    )(_bias_body)
    return k3(parts[0], parts[1], parts[2], parts[3], da, db, bias_flat,
              tail_bias, dense_w, dense_b)


def kernel(drug_a, drug_b, emb_table, bias_table, dense_w, dense_b):
    da = drug_a.astype(jnp.int32)
    db = drug_b.astype(jnp.int32)
    emb_t = emb_table.T
    flats = []
    src = emb_t
    for g in range(NG):
        if flats:
            src, _ = lax.optimization_barrier((emb_t, flats[-1]))
        flats.append(src[g * NF:(g + 1) * NF, :NMAIN]
                     .reshape(NF // 8, 8, NB, 128)
                     .transpose(0, 2, 1, 3).reshape(FLATG))
    lastflat, bias_table = lax.optimization_barrier((flats[-1], bias_table))
    flats[-1] = lastflat
    bias_flat = bias_table.T[:, :NMAIN].reshape(NMAIN)
    tail_bias = bias_table.T[:, NMAIN:NMAIN + D].reshape(D)
    tail = emb_table[NMAIN:NMAIN + D]
    w_pad = jnp.pad(dense_w.reshape(-1), (0, L - 1))
    b0_pad = jnp.pad(dense_b, (0, L - 1))
    out = _afmp(da, db, flats, tail, bias_flat, tail_bias, w_pad, b0_pad)
    return out.reshape(B, 1)
